# baseline (speedup 1.0000x reference)
import jax
import jax.numpy as jnp
from jax import lax
from jax.experimental import pallas as pl
from jax.experimental.pallas import tpu as pltpu
from jax.experimental.pallas import tpu_sc as plsc

N = 320000
D = 128
S = 10000

NC = 2
NS = 16
NW = NC * NS
CHUNK = N // NW
SP = 10240
SLICE = SP // NS
L = 16

BROWS = 16000
BQ = BROWS // 4
NB = N // BROWS


def _tc_body(x0_ref, x1_ref, x2_ref, x3_ref, w_ref, o_ref):
    wv = w_ref[...]
    for k, x_ref in enumerate((x0_ref, x1_ref, x2_ref, x3_ref)):
        x = x_ref[...]
        z = lax.dot_general(wv, x, dimension_numbers=(((1,), (1,)), ((), ())),
                            preferred_element_type=jnp.float32)
        o_ref[0, :, k * BQ:(k + 1) * BQ] = jnp.exp(z)


def _sc_sums_body(y_hbm, ids_hbm, partial_hbm, ids_v, y_v, sums_v,
                  spmem_all, tmp_v, acc_v):
    c = lax.axis_index("c")
    s = lax.axis_index("s")
    w = c * NS + s
    base = w * CHUNK
    pltpu.sync_copy(ids_hbm.at[pl.ds(base, CHUNK)], ids_v)
    pltpu.sync_copy(y_hbm.at[pl.ds(base, CHUNK)], y_v)

    @plsc.parallel_loop(0, SP // L, unroll=4)
    def _zero(i):
        off = pl.multiple_of(i * L, L)
        sums_v[pl.ds(off, L)] = jnp.zeros((L,), jnp.float32)

    UN = 5

    def scat_body(i, carry):
        base_off = pl.multiple_of(i * (L * UN), L)
        for k in range(UN):
            off = base_off + k * L
            ids16 = ids_v[pl.ds(off, L)]
            y16 = y_v[pl.ds(off, L)]
            plsc.addupdate_scatter(sums_v, [ids16], y16)
        return carry
    lax.fori_loop(0, CHUNK // (L * UN), scat_body, 0)

    pltpu.sync_copy(sums_v, spmem_all.at[s])
    plsc.subcore_barrier()

    seg0 = pl.multiple_of(s * SLICE, L)
    pltpu.sync_copy(spmem_all.at[:, pl.ds(seg0, SLICE)], tmp_v)

    @plsc.parallel_loop(0, SLICE // L, unroll=2)
    def _combine(i):
        off = pl.multiple_of(i * L, L)
        acc = tmp_v[0, pl.ds(off, L)]
        for t in range(1, NS):
            acc = acc + tmp_v[t, pl.ds(off, L)]
        acc_v[pl.ds(off, L)] = acc

    pltpu.sync_copy(acc_v, partial_hbm.at[c, pl.ds(seg0, SLICE)])


def _sc_norm_body(y_hbm, ids_hbm, partial_hbm, out_hbm, ids_v, y_v,
                  p0_v, p1_v, out_v):
    c = lax.axis_index("c")
    s = lax.axis_index("s")
    w = c * NS + s
    base = w * CHUNK
    pltpu.sync_copy(ids_hbm.at[pl.ds(base, CHUNK)], ids_v)
    pltpu.sync_copy(y_hbm.at[pl.ds(base, CHUNK)], y_v)
    pltpu.sync_copy(partial_hbm.at[0], p0_v)
    pltpu.sync_copy(partial_hbm.at[1], p1_v)

    @plsc.parallel_loop(0, SP // L, unroll=4)
    def _padd(i):
        off = pl.multiple_of(i * L, L)
        p0_v[pl.ds(off, L)] = p0_v[pl.ds(off, L)] + p1_v[pl.ds(off, L)]

    @plsc.parallel_loop(0, CHUNK // L, unroll=4)
    def _norm(i):
        off = pl.multiple_of(i * L, L)
        ids16 = ids_v[pl.ds(off, L)]
        denom = plsc.load_gather(p0_v, [ids16])
        out_v[pl.ds(off, L)] = y_v[pl.ds(off, L)] / denom

    pltpu.sync_copy(out_v, out_hbm.at[pl.ds(base, CHUNK)])


_SC_KERNELS = None


def _sc_kernels():
    global _SC_KERNELS
    if _SC_KERNELS is None:
        mesh = plsc.VectorSubcoreMesh(
            core_axis_name="c", subcore_axis_name="s",
            num_cores=NC, num_subcores=NS)
        sc_params = pltpu.CompilerParams(needs_layout_passes=False)
        sums = pl.kernel(
            _sc_sums_body,
            out_type=jax.ShapeDtypeStruct((NC, SP), jnp.float32),
            mesh=mesh,
            compiler_params=sc_params,
            scratch_types=[
                pltpu.VMEM((CHUNK,), jnp.int32),
                pltpu.VMEM((CHUNK,), jnp.float32),
                pltpu.VMEM((SP,), jnp.float32),
                pltpu.VMEM_SHARED((NS, SP), jnp.float32),
                pltpu.VMEM((NS, SLICE), jnp.float32),
                pltpu.VMEM((SLICE,), jnp.float32),
            ],
        )
        norm = pl.kernel(
            _sc_norm_body,
            out_type=jax.ShapeDtypeStruct((N,), jnp.float32),
            mesh=mesh,
            compiler_params=sc_params,
            scratch_types=[
                pltpu.VMEM((CHUNK,), jnp.int32),
                pltpu.VMEM((CHUNK,), jnp.float32),
                pltpu.VMEM((SP,), jnp.float32),
                pltpu.VMEM((SP,), jnp.float32),
                pltpu.VMEM((CHUNK,), jnp.float32),
            ],
        )
        _SC_KERNELS = (sums, norm)
    return _SC_KERNELS

_tc_exp_matvec = pl.pallas_call(
    _tc_body,
    grid=(NB,),
    in_specs=[
        pl.BlockSpec((BQ, D), lambda i: (4 * i, 0)),
        pl.BlockSpec((BQ, D), lambda i: (4 * i + 1, 0)),
        pl.BlockSpec((BQ, D), lambda i: (4 * i + 2, 0)),
        pl.BlockSpec((BQ, D), lambda i: (4 * i + 3, 0)),
        pl.BlockSpec((1, D), lambda i: (0, 0)),
    ],
    out_specs=pl.BlockSpec((1, 1, BROWS), lambda i: (i, 0, 0)),
    out_shape=jax.ShapeDtypeStruct((NB, 1, BROWS), jnp.float32),
    compiler_params=pltpu.CompilerParams(
        dimension_semantics=("parallel",)),
)


def kernel(X, segment_ids, W, b):
    del b
    sc_sums, sc_norm = _sc_kernels()
    y = _tc_exp_matvec(X, X, X, X, W).reshape(N)
    ids = segment_ids.astype(jnp.int32)
    partial = sc_sums(y, ids)
    out = sc_norm(y, ids, partial)
    return out.reshape(N, 1)


# speedup vs baseline: 1.0089x; 1.0089x over previous
import jax
import jax.numpy as jnp
from jax import lax
from jax.experimental import pallas as pl
from jax.experimental.pallas import tpu as pltpu
from jax.experimental.pallas import tpu_sc as plsc

N = 320000
D = 128
S = 10000

NC = 2
NS = 16
NW = NC * NS
CHUNK = N // NW
SP = 10240
SLICE = SP // NS
L = 16

BROWS = 16000
BQ = BROWS // 4
NB = N // BROWS


def _tc_body(x_ref, w_ref, o_ref):
    x = x_ref[...]
    wv = w_ref[...]
    z = lax.dot_general(wv, x, dimension_numbers=(((1,), (1,)), ((), ())),
                        preferred_element_type=jnp.float32)
    o_ref[0] = jnp.exp(z)


def _sc_sums_body(y_hbm, ids_hbm, partial_hbm, ids_v, y_v, sums_v,
                  spmem_all, tmp_v, acc_v):
    c = lax.axis_index("c")
    s = lax.axis_index("s")
    w = c * NS + s
    base = w * CHUNK
    pltpu.sync_copy(ids_hbm.at[pl.ds(base, CHUNK)], ids_v)
    pltpu.sync_copy(y_hbm.at[pl.ds(base, CHUNK)], y_v)

    @plsc.parallel_loop(0, SP // L, unroll=4)
    def _zero(i):
        off = pl.multiple_of(i * L, L)
        sums_v[pl.ds(off, L)] = jnp.zeros((L,), jnp.float32)

    UN = 5

    def scat_body(i, carry):
        base_off = pl.multiple_of(i * (L * UN), L)
        for k in range(UN):
            off = base_off + k * L
            ids16 = ids_v[pl.ds(off, L)]
            y16 = y_v[pl.ds(off, L)]
            plsc.addupdate_scatter(sums_v, [ids16], y16)
        return carry
    lax.fori_loop(0, CHUNK // (L * UN), scat_body, 0)

    pltpu.sync_copy(sums_v, spmem_all.at[s])
    plsc.subcore_barrier()

    seg0 = pl.multiple_of(s * SLICE, L)
    pltpu.sync_copy(spmem_all.at[:, pl.ds(seg0, SLICE)], tmp_v)

    @plsc.parallel_loop(0, SLICE // L, unroll=2)
    def _combine(i):
        off = pl.multiple_of(i * L, L)
        acc = tmp_v[0, pl.ds(off, L)]
        for t in range(1, NS):
            acc = acc + tmp_v[t, pl.ds(off, L)]
        acc_v[pl.ds(off, L)] = acc

    pltpu.sync_copy(acc_v, partial_hbm.at[c, pl.ds(seg0, SLICE)])


def _sc_norm_body(y_hbm, ids_hbm, partial_hbm, out_hbm, ids_v, y_v,
                  p0_v, p1_v, out_v):
    c = lax.axis_index("c")
    s = lax.axis_index("s")
    w = c * NS + s
    base = w * CHUNK
    pltpu.sync_copy(ids_hbm.at[pl.ds(base, CHUNK)], ids_v)
    pltpu.sync_copy(y_hbm.at[pl.ds(base, CHUNK)], y_v)
    pltpu.sync_copy(partial_hbm.at[0], p0_v)
    pltpu.sync_copy(partial_hbm.at[1], p1_v)

    @plsc.parallel_loop(0, SP // L, unroll=4)
    def _padd(i):
        off = pl.multiple_of(i * L, L)
        p0_v[pl.ds(off, L)] = p0_v[pl.ds(off, L)] + p1_v[pl.ds(off, L)]

    @plsc.parallel_loop(0, CHUNK // L, unroll=4)
    def _norm(i):
        off = pl.multiple_of(i * L, L)
        ids16 = ids_v[pl.ds(off, L)]
        denom = plsc.load_gather(p0_v, [ids16])
        out_v[pl.ds(off, L)] = y_v[pl.ds(off, L)] / denom

    pltpu.sync_copy(out_v, out_hbm.at[pl.ds(base, CHUNK)])


_SC_KERNELS = None


def _sc_kernels():
    global _SC_KERNELS
    if _SC_KERNELS is None:
        mesh = plsc.VectorSubcoreMesh(
            core_axis_name="c", subcore_axis_name="s",
            num_cores=NC, num_subcores=NS)
        sc_params = pltpu.CompilerParams(needs_layout_passes=False)
        sums = pl.kernel(
            _sc_sums_body,
            out_type=jax.ShapeDtypeStruct((NC, SP), jnp.float32),
            mesh=mesh,
            compiler_params=sc_params,
            scratch_types=[
                pltpu.VMEM((CHUNK,), jnp.int32),
                pltpu.VMEM((CHUNK,), jnp.float32),
                pltpu.VMEM((SP,), jnp.float32),
                pltpu.VMEM_SHARED((NS, SP), jnp.float32),
                pltpu.VMEM((NS, SLICE), jnp.float32),
                pltpu.VMEM((SLICE,), jnp.float32),
            ],
        )
        norm = pl.kernel(
            _sc_norm_body,
            out_type=jax.ShapeDtypeStruct((N,), jnp.float32),
            mesh=mesh,
            compiler_params=sc_params,
            scratch_types=[
                pltpu.VMEM((CHUNK,), jnp.int32),
                pltpu.VMEM((CHUNK,), jnp.float32),
                pltpu.VMEM((SP,), jnp.float32),
                pltpu.VMEM((SP,), jnp.float32),
                pltpu.VMEM((CHUNK,), jnp.float32),
            ],
        )
        _SC_KERNELS = (sums, norm)
    return _SC_KERNELS

_tc_exp_matvec = pl.pallas_call(
    _tc_body,
    grid=(NB,),
    in_specs=[
        pl.BlockSpec((BROWS, D), lambda i: (i, 0)),
        pl.BlockSpec((1, D), lambda i: (0, 0)),
    ],
    out_specs=pl.BlockSpec((1, 1, BROWS), lambda i: (i, 0, 0)),
    out_shape=jax.ShapeDtypeStruct((NB, 1, BROWS), jnp.float32),
    compiler_params=pltpu.CompilerParams(
        dimension_semantics=("parallel",)),
)


def kernel(X, segment_ids, W, b):
    del b
    sc_sums, sc_norm = _sc_kernels()
    y = _tc_exp_matvec(X, W).reshape(N)
    ids = segment_ids.astype(jnp.int32)
    partial = sc_sums(y, ids)
    out = sc_norm(y, ids, partial)
    return out.reshape(N, 1)
